# baseline (speedup 1.0000x reference)
import functools

import jax
import jax.numpy as jnp
import numpy as np
from jax import lax
from jax.experimental import pallas as pl
from jax.experimental.pallas import tpu as pltpu
from jax.experimental.pallas import tpu_sc as plsc

_ROWS_PER_DMA = 128

_PERM = np.empty(128, np.int32)
for _g in range(4):
    for _k in range(16):
        _PERM[_g * 32 + 2 * _k] = _g * 32 + _k
        _PERM[_g * 32 + 2 * _k + 1] = _g * 32 + 16 + _k
_INV_PERM = np.argsort(_PERM).astype(np.int32)


def _sc_gather_bf16(table, idx3d):
    nw, ch, rpd = idx3d.shape
    _, d = table.shape
    dw = d // 2
    info = plsc.get_sparse_core_info()
    nc = info.num_cores
    nbuf = min(3, ch)
    assert ch <= 20

    mesh = plsc.VectorSubcoreMesh(core_axis_name="c", subcore_axis_name="s")

    @functools.partial(
        pl.kernel,
        mesh=mesh,
        out_type=jax.ShapeDtypeStruct((nw * ch * rpd, dw), jnp.int32),
        scratch_types=[pltpu.VMEM((ch, rpd), jnp.int32)]
        + [pltpu.VMEM((rpd, d), jnp.float32) for _ in range(nbuf)]
        + [pltpu.VMEM((rpd, dw), jnp.int32) for _ in range(nbuf)]
        + [pltpu.SemaphoreType.DMA for _ in range(2 * nbuf)],
    )
    def gather_kernel(table_hbm, idx_hbm, e_hbm, idx_v, *scratch):
        bufs = scratch[:nbuf]
        cbufs = scratch[nbuf:2 * nbuf]
        gsems = scratch[2 * nbuf:3 * nbuf]
        wsems = scratch[3 * nbuf:]
        wid = lax.axis_index("s") * nc + lax.axis_index("c")
        pltpu.sync_copy(idx_hbm.at[wid], idx_v)
        rbase = wid * ch * rpd

        def g(j, k):
            return pltpu.make_async_copy(table_hbm.at[idx_v.at[j]], bufs[k],
                                         gsems[k])

        def w(j, k):
            return pltpu.make_async_copy(
                cbufs[k], e_hbm.at[pl.ds(rbase + j * rpd, rpd)], wsems[k])

        def convert(k):
            buf, cbuf = bufs[k], cbufs[k]
            hi_mask = jnp.full((16,), -65536, jnp.int32)
            sh16 = jnp.full((16,), 16, jnp.int32)

            def rows(r0, carry):
                for rr in range(8):
                    r = r0 * 8 + rr
                    for gi in range(4):
                        a = lax.bitcast_convert_type(
                            buf[r, pl.ds(gi * 32, 16)], jnp.int32)
                        bv = lax.bitcast_convert_type(
                            buf[r, pl.ds(gi * 32 + 16, 16)], jnp.int32)
                        cbuf[r, pl.ds(gi * 16, 16)] = lax.bitwise_or(
                            lax.shift_right_logical(a, sh16),
                            lax.bitwise_and(bv, hi_mask))
                return carry

            lax.fori_loop(0, rpd // 8, rows, 0)

        for j in range(nbuf):
            g(j, j).start()
        for j in range(ch):
            k = j % nbuf
            g(j, k).wait()
            convert(k)
            w(j, k).start()
            if j + nbuf < ch:
                w(j, k).wait()
                g(j + nbuf, k).start()
        for j in range(max(0, ch - nbuf), ch):
            w(j, j % nbuf).wait()

    return gather_kernel(table, idx3d)


def _tc_body(bb, tt, tok_base, len_ref, e_ref, w_ref, b_ref, out_ref):
    lblk = pl.program_id(1)
    ln = len_ref[...]
    inv = 1.0 / jnp.maximum(ln, 1).astype(jnp.float32)
    wmat = w_ref[...]
    bias = b_ref[...]
    acc = jnp.zeros(out_ref.shape, jnp.float32)
    for t in range(tt):
        tok = tok_base + lblk * tt + t
        e = e_ref[t].astype(jnp.float32)
        h = e + jnp.maximum(
            jnp.dot(e, wmat, preferred_element_type=jnp.float32) + bias, 0.0)
        m = (tok < ln).astype(jnp.float32)
        acc = acc + h * m
    contrib = acc * inv

    @pl.when(lblk == 0)
    def _():
        out_ref[...] = contrib

    @pl.when(lblk > 0)
    def _():
        out_ref[...] += contrib


def _tc_mean(lengths_col, e3, wmat, brow, bb, tt, tok_base=0):
    ltot, b, d = e3.shape
    grid = (b // bb, ltot // tt)
    return pl.pallas_call(
        functools.partial(_tc_body, bb, tt, tok_base),
        grid=grid,
        in_specs=[
            pl.BlockSpec((bb, 1), lambda i, l: (i, 0)),
            pl.BlockSpec((tt, bb, d), lambda i, l: (l, i, 0)),
            pl.BlockSpec((d, d), lambda i, l: (0, 0)),
            pl.BlockSpec((1, d), lambda i, l: (0, 0)),
        ],
        out_specs=pl.BlockSpec((bb, d), lambda i, l: (i, 0)),
        out_shape=jax.ShapeDtypeStruct((b, d), jnp.float32),
    )(lengths_col, e3, wmat, brow)


_TOKEN_CHUNKS = 5


def kernel(x, initialHidden, lengths, table, W, b):
    del initialHidden
    bsz, seq = x.shape
    _, d = table.shape
    info = plsc.get_sparse_core_info()
    nw = info.num_cores * info.num_subcores
    xt = x.T.astype(jnp.int32)
    lcol = lengths.astype(jnp.int32).reshape(bsz, 1)
    perm = jnp.asarray(_PERM)
    invp = jnp.asarray(_INV_PERM)
    wt = W.T[perm][:, perm]
    brow = b[perm].reshape(1, d)

    ltok = seq // _TOKEN_CHUNKS
    partials = []
    for c in range(_TOKEN_CHUNKS):
        idx3d = xt[c * ltok:(c + 1) * ltok].reshape(nw, -1, _ROWS_PER_DMA)
        e_i32 = _sc_gather_bf16(table, idx3d)
        e_bf = jax.lax.bitcast_convert_type(e_i32, jnp.bfloat16)
        e3 = e_bf.reshape(ltok, bsz, d)
        partials.append(
            _tc_mean(lcol, e3, wt, brow, bb=1024, tt=ltok, tok_base=c * ltok))
    out = partials[0]
    for p in partials[1:]:
        out = out + p
    return out[:, invp]


# speedup vs baseline: 1.0033x; 1.0033x over previous
import functools

import jax
import jax.numpy as jnp
import numpy as np
from jax import lax
from jax.experimental import pallas as pl
from jax.experimental.pallas import tpu as pltpu
from jax.experimental.pallas import tpu_sc as plsc

_ROWS_PER_DMA = 128

_PERM = np.empty(128, np.int32)
for _g in range(4):
    for _k in range(16):
        _PERM[_g * 32 + 2 * _k] = _g * 32 + _k
        _PERM[_g * 32 + 2 * _k + 1] = _g * 32 + 16 + _k
_INV_PERM = np.argsort(_PERM).astype(np.int32)


def _sc_gather_bf16(table, idx3d):
    nw, ch, rpd = idx3d.shape
    _, d = table.shape
    dw = d // 2
    info = plsc.get_sparse_core_info()
    nc = info.num_cores
    nbuf = min(3, ch)
    assert ch <= 20

    mesh = plsc.VectorSubcoreMesh(core_axis_name="c", subcore_axis_name="s")

    @functools.partial(
        pl.kernel,
        mesh=mesh,
        out_type=jax.ShapeDtypeStruct((nw * ch * rpd, dw), jnp.int32),
        scratch_types=[pltpu.VMEM((ch, rpd), jnp.int32)]
        + [pltpu.VMEM((rpd, d), jnp.float32) for _ in range(nbuf)]
        + [pltpu.VMEM((rpd, dw), jnp.int32) for _ in range(nbuf)]
        + [pltpu.SemaphoreType.DMA for _ in range(2 * nbuf)],
    )
    def gather_kernel(table_hbm, idx_hbm, e_hbm, idx_v, *scratch):
        bufs = scratch[:nbuf]
        cbufs = scratch[nbuf:2 * nbuf]
        gsems = scratch[2 * nbuf:3 * nbuf]
        wsems = scratch[3 * nbuf:]
        wid = lax.axis_index("s") * nc + lax.axis_index("c")
        pltpu.sync_copy(idx_hbm.at[wid], idx_v)
        rbase = wid * ch * rpd

        def g(j, k):
            return pltpu.make_async_copy(table_hbm.at[idx_v.at[j]], bufs[k],
                                         gsems[k])

        def w(j, k):
            return pltpu.make_async_copy(
                cbufs[k], e_hbm.at[pl.ds(rbase + j * rpd, rpd)], wsems[k])

        def convert(k):
            buf, cbuf = bufs[k], cbufs[k]
            hi_mask = jnp.full((16,), -65536, jnp.int32)
            sh16 = jnp.full((16,), 16, jnp.int32)

            @plsc.parallel_loop(0, rpd, unroll=8)
            def _(r):
                for gi in range(4):
                    a = lax.bitcast_convert_type(
                        buf[r, pl.ds(gi * 32, 16)], jnp.int32)
                    bv = lax.bitcast_convert_type(
                        buf[r, pl.ds(gi * 32 + 16, 16)], jnp.int32)
                    cbuf[r, pl.ds(gi * 16, 16)] = lax.bitwise_or(
                        lax.shift_right_logical(a, sh16),
                        lax.bitwise_and(bv, hi_mask))

        for j in range(nbuf):
            g(j, j).start()
        for j in range(ch):
            k = j % nbuf
            g(j, k).wait()
            convert(k)
            w(j, k).start()
            if j + nbuf < ch:
                w(j, k).wait()
                g(j + nbuf, k).start()
        for j in range(max(0, ch - nbuf), ch):
            w(j, j % nbuf).wait()

    return gather_kernel(table, idx3d)


def _tc_body(bb, tt, tok_base, len_ref, e_ref, w_ref, b_ref, out_ref):
    lblk = pl.program_id(1)
    ln = len_ref[...]
    inv = 1.0 / jnp.maximum(ln, 1).astype(jnp.float32)
    wmat = w_ref[...]
    bias = b_ref[...]
    acc = jnp.zeros(out_ref.shape, jnp.float32)
    for t in range(tt):
        tok = tok_base + lblk * tt + t
        e = e_ref[t].astype(jnp.float32)
        h = e + jnp.maximum(
            jnp.dot(e, wmat, preferred_element_type=jnp.float32) + bias, 0.0)
        m = (tok < ln).astype(jnp.float32)
        acc = acc + h * m
    contrib = acc * inv

    @pl.when(lblk == 0)
    def _():
        out_ref[...] = contrib

    @pl.when(lblk > 0)
    def _():
        out_ref[...] += contrib


def _tc_mean(lengths_col, e3, wmat, brow, bb, tt, tok_base=0):
    ltot, b, d = e3.shape
    grid = (b // bb, ltot // tt)
    return pl.pallas_call(
        functools.partial(_tc_body, bb, tt, tok_base),
        grid=grid,
        in_specs=[
            pl.BlockSpec((bb, 1), lambda i, l: (i, 0)),
            pl.BlockSpec((tt, bb, d), lambda i, l: (l, i, 0)),
            pl.BlockSpec((d, d), lambda i, l: (0, 0)),
            pl.BlockSpec((1, d), lambda i, l: (0, 0)),
        ],
        out_specs=pl.BlockSpec((bb, d), lambda i, l: (i, 0)),
        out_shape=jax.ShapeDtypeStruct((b, d), jnp.float32),
    )(lengths_col, e3, wmat, brow)


_TOKEN_CHUNKS = 5


def kernel(x, initialHidden, lengths, table, W, b):
    del initialHidden
    bsz, seq = x.shape
    _, d = table.shape
    info = plsc.get_sparse_core_info()
    nw = info.num_cores * info.num_subcores
    xt = x.T.astype(jnp.int32)
    lcol = lengths.astype(jnp.int32).reshape(bsz, 1)
    perm = jnp.asarray(_PERM)
    invp = jnp.asarray(_INV_PERM)
    wt = W.T[perm][:, perm]
    brow = b[perm].reshape(1, d)

    ltok = seq // _TOKEN_CHUNKS
    partials = []
    for c in range(_TOKEN_CHUNKS):
        idx3d = xt[c * ltok:(c + 1) * ltok].reshape(nw, -1, _ROWS_PER_DMA)
        e_i32 = _sc_gather_bf16(table, idx3d)
        e_bf = jax.lax.bitcast_convert_type(e_i32, jnp.bfloat16)
        e3 = e_bf.reshape(ltok, bsz, d)
        partials.append(
            _tc_mean(lcol, e3, wt, brow, bb=1024, tt=ltok, tok_base=c * ltok))
    out = partials[0]
    for p in partials[1:]:
        out = out + p
    return out[:, invp]


# speedup vs baseline: 4.1345x; 4.1211x over previous
import functools

import jax
import jax.numpy as jnp
from jax import lax
from jax.experimental import pallas as pl
from jax.experimental.pallas import tpu as pltpu
from jax.experimental.pallas import tpu_sc as plsc

_ROWS_PER_DMA = 128


def _sc_gather(table, idx3d):
    nw, ch, rpd = idx3d.shape
    nchunks = nw * ch
    _, d = table.shape
    info = plsc.get_sparse_core_info()
    nc = info.num_cores

    mesh = plsc.VectorSubcoreMesh(core_axis_name="c", subcore_axis_name="s")

    nbuf = min(4, ch)
    assert ch <= 20, "keep the static unroll well under the TileTask size cap"

    @functools.partial(
        pl.kernel,
        mesh=mesh,
        out_type=jax.ShapeDtypeStruct((nchunks * rpd, d), jnp.float32),
        scratch_types=[pltpu.VMEM((ch, rpd), jnp.int32)]
        + [pltpu.VMEM((rpd, d), jnp.float32) for _ in range(nbuf)]
        + [pltpu.SemaphoreType.DMA for _ in range(2 * nbuf)],
    )
    def gather_kernel(table_hbm, idx_hbm, e_hbm, idx_v, *scratch):
        bufs = scratch[:nbuf]
        gsems = scratch[nbuf:2 * nbuf]
        wsems = scratch[2 * nbuf:]
        wid = lax.axis_index("s") * nc + lax.axis_index("c")
        pltpu.sync_copy(idx_hbm.at[wid], idx_v)
        rbase = wid * ch * rpd

        def g(j, k):
            return pltpu.make_async_copy(table_hbm.at[idx_v.at[j]], bufs[k],
                                         gsems[k])

        def w(j, k):
            return pltpu.make_async_copy(
                bufs[k], e_hbm.at[pl.ds(rbase + j * rpd, rpd)], wsems[k])

        for j in range(nbuf):
            g(j, j).start()
        for j in range(ch):
            k = j % nbuf
            g(j, k).wait()
            w(j, k).start()
            if j + nbuf < ch:
                w(j, k).wait()
                g(j + nbuf, k).start()
        for j in range(max(0, ch - nbuf), ch):
            w(j, j % nbuf).wait()

    return gather_kernel(table, idx3d)


def _tc_body(bb, tt, tok_base, len_ref, e_ref, w_ref, b_ref, out_ref):
    lblk = pl.program_id(1)
    ln = len_ref[...]
    inv = 1.0 / jnp.maximum(ln, 1).astype(jnp.float32)
    wmat = w_ref[...]
    bias = b_ref[...]
    acc = jnp.zeros(out_ref.shape, jnp.float32)
    for t in range(tt):
        tok = tok_base + lblk * tt + t
        e = e_ref[t]
        h = e + jnp.maximum(
            jnp.dot(e, wmat, preferred_element_type=jnp.float32) + bias, 0.0
        )
        m = (tok < ln).astype(jnp.float32)
        acc = acc + h * m
    contrib = acc * inv

    @pl.when(lblk == 0)
    def _():
        out_ref[...] = contrib

    @pl.when(lblk > 0)
    def _():
        out_ref[...] += contrib


def _tc_mean(lengths_col, e3, wmat, brow, bb, tt, tok_base=0):
    ltot, b, d = e3.shape
    grid = (b // bb, ltot // tt)
    return pl.pallas_call(
        functools.partial(_tc_body, bb, tt, tok_base),
        grid=grid,
        in_specs=[
            pl.BlockSpec((bb, 1), lambda i, l: (i, 0)),
            pl.BlockSpec((tt, bb, d), lambda i, l: (l, i, 0)),
            pl.BlockSpec((d, d), lambda i, l: (0, 0)),
            pl.BlockSpec((1, d), lambda i, l: (0, 0)),
        ],
        out_specs=pl.BlockSpec((bb, d), lambda i, l: (i, 0)),
        out_shape=jax.ShapeDtypeStruct((b, d), jnp.float32),
    )(lengths_col, e3, wmat, brow)


_TOKEN_CHUNKS = 5


def kernel(x, initialHidden, lengths, table, W, b):
    del initialHidden
    bsz, seq = x.shape
    _, d = table.shape
    info = plsc.get_sparse_core_info()
    nw = info.num_cores * info.num_subcores
    xt = x.T.astype(jnp.int32)
    lcol = lengths.astype(jnp.int32).reshape(bsz, 1)
    wt = W.T
    brow = b.reshape(1, d)

    ltok = seq // _TOKEN_CHUNKS
    partials = []
    for c in range(_TOKEN_CHUNKS):
        idx3d = xt[c * ltok:(c + 1) * ltok].reshape(nw, -1, _ROWS_PER_DMA)
        e_flat = _sc_gather(table, idx3d)
        e3 = e_flat.reshape(ltok, bsz, d)
        partials.append(
            _tc_mean(lcol, e3, wt, brow, bb=1024, tt=ltok, tok_base=c * ltok))
    out = partials[0]
    for p in partials[1:]:
        out = out + p
    return out
